# Initial kernel scaffold; baseline (speedup 1.0000x reference)
import jax
import jax.numpy as jnp
from jax.experimental import pallas as pl


def kernel(x_var, edge_index, clue_values, num_con, W_msg, b_msg, W_upd, b_upd, gamma, beta):
    raise NotImplementedError("write your pallas kernel here")


# speedup vs baseline: 4.4417x; 4.4417x over previous
import dataclasses
import functools

import jax
import jax.numpy as jnp
from jax import lax
from jax.experimental import pallas as pl
from jax.experimental.pallas import tpu as pltpu
from jax.experimental.pallas import tpu_sc as plsc

NC = 2
NS = 16
NW = NC * NS
CH = 128


def _sc_segment_sum(x_var, src3, dst3, z_acc, z_cnt,
                    n_chunks, n_acc, rows_per_sub, H):
    mesh = plsc.VectorSubcoreMesh(core_axis_name="c", subcore_axis_name="s",
                                  num_cores=NC, num_subcores=NS)
    cp = pltpu.CompilerParams()
    if "needs_layout_passes" in pltpu.CompilerParams.__dataclass_fields__:
        cp = dataclasses.replace(cp, needs_layout_passes=False)

    @functools.partial(
        pl.kernel,
        compiler_params=cp,
        out_type=(
            jax.ShapeDtypeStruct((NC, NS, rows_per_sub, H), jnp.float32),
            jax.ShapeDtypeStruct((NW, n_acc), jnp.float32),
        ),
        mesh=mesh,
        scratch_types=[
            pltpu.VMEM((n_chunks, CH), jnp.int32),
            pltpu.VMEM((n_chunks, CH), jnp.int32),
            pltpu.VMEM((CH, H), jnp.float32),
            pltpu.VMEM((n_acc,), jnp.float32),
            pltpu.VMEM_SHARED((n_acc, H), jnp.float32),
            pltpu.SemaphoreType.DMA,
        ],
    )
    def sc_kernel(x_hbm, src_hbm, dst_hbm, zacc_hbm, zcnt_hbm,
                  acc_hbm, cnt_hbm,
                  src_v, dst_v, rows_v, cnt_v, acc_sh, sem):
        cid = lax.axis_index("c")
        sid = lax.axis_index("s")
        wid = sid * NC + cid
        pltpu.sync_copy(src_hbm.at[wid], src_v)
        pltpu.sync_copy(dst_hbm.at[wid], dst_v)
        pltpu.sync_copy(zcnt_hbm, cnt_v)
        row0 = sid * rows_per_sub
        pltpu.sync_copy(zacc_hbm, acc_sh.at[pl.ds(row0, rows_per_sub)])
        plsc.subcore_barrier()

        ones_reg = jnp.ones((16,), jnp.float32)

        @pl.loop(0, n_chunks)
        def _(j):
            pltpu.async_copy(x_hbm.at[src_v.at[j]], rows_v, sem).wait()
            pltpu.sync_copy(rows_v, acc_sh.at[dst_v.at[j]], add=True)

            @pl.loop(0, CH // 16)
            def _(k):
                idx = dst_v[j, pl.ds(k * 16, 16)]
                plsc.addupdate_scatter(cnt_v, [idx], ones_reg)

        plsc.subcore_barrier()
        pltpu.sync_copy(acc_sh.at[pl.ds(row0, rows_per_sub)], acc_hbm.at[cid, sid])
        pltpu.sync_copy(cnt_v, cnt_hbm.at[wid])

    return sc_kernel(x_var, src3, dst3, z_acc, z_cnt)


def _tail_body(acc_ref, cnt_ref, clue_ref, wm_ref, bm_ref, wua_ref, wc_ref,
               bu_ref, g_ref, be_ref, o_ref):
    A = acc_ref[0] + acc_ref[1]
    cnt = jnp.sum(cnt_ref[...], axis=1, keepdims=True)
    m = lax.dot_general(A, wm_ref[...], (((1,), (0,)), ((), ())),
                        precision=lax.Precision.HIGHEST)
    agg = (m + cnt * bm_ref[...]) / (cnt + 1e-6)
    u = lax.dot_general(agg, wua_ref[...], (((1,), (0,)), ((), ())),
                        precision=lax.Precision.HIGHEST)
    u = u + clue_ref[...] * wc_ref[...] + bu_ref[...]
    u = jnp.maximum(u, 0.0)
    mu = jnp.mean(u, axis=1, keepdims=True)
    var = jnp.mean((u - mu) ** 2, axis=1, keepdims=True)
    o_ref[...] = (u - mu) * lax.rsqrt(var + 1e-5) * g_ref[...] + be_ref[...]


def kernel(x_var, edge_index, clue_values, num_con,
           W_msg, b_msg, W_upd, b_upd, gamma, beta):
    N_var, H = x_var.shape
    N_con = clue_values.shape[0]
    E = edge_index.shape[1]
    src = edge_index[0].astype(jnp.int32)
    dst = edge_index[1].astype(jnp.int32)

    n_chunks = -(-E // (NW * CH))
    e_pad = NW * CH * n_chunks - E
    if e_pad:
        src = jnp.concatenate([src, jnp.zeros((e_pad,), jnp.int32)])
        dst = jnp.concatenate([dst, jnp.full((e_pad,), N_con, jnp.int32)])
    src3 = src.reshape(NW, n_chunks, CH)
    dst3 = dst.reshape(NW, n_chunks, CH)

    rows_per_sub = (-(-(N_con + 1) // NS) + 7) // 8 * 8
    n_acc = rows_per_sub * NS

    z_acc = jnp.zeros((rows_per_sub, H), jnp.float32)
    z_cnt = jnp.zeros((n_acc,), jnp.float32)

    acc, cnt = _sc_segment_sum(x_var, src3, dst3, z_acc, z_cnt,
                               n_chunks, n_acc, rows_per_sub, H)
    acc = acc.reshape(NC, n_acc, H)
    cnt_t = cnt.T

    delta = (jnp.asarray(num_con) - N_con).astype(jnp.float32)
    beta_eff = (beta + delta).reshape(1, H)

    BLK = 1000
    grid = -(-N_con // BLK)
    out = pl.pallas_call(
        _tail_body,
        grid=(grid,),
        in_specs=[
            pl.BlockSpec((NC, BLK, H), lambda i: (0, i, 0)),
            pl.BlockSpec((BLK, NW), lambda i: (i, 0)),
            pl.BlockSpec((BLK, 1), lambda i: (i, 0)),
            pl.BlockSpec((H, H), lambda i: (0, 0)),
            pl.BlockSpec((1, H), lambda i: (0, 0)),
            pl.BlockSpec((H, H), lambda i: (0, 0)),
            pl.BlockSpec((1, H), lambda i: (0, 0)),
            pl.BlockSpec((1, H), lambda i: (0, 0)),
            pl.BlockSpec((1, H), lambda i: (0, 0)),
            pl.BlockSpec((1, H), lambda i: (0, 0)),
        ],
        out_specs=pl.BlockSpec((BLK, H), lambda i: (i, 0)),
        out_shape=jax.ShapeDtypeStruct((N_con, H), jnp.float32),
    )(acc, cnt_t, clue_values.reshape(N_con, 1), W_msg, b_msg.reshape(1, H),
      W_upd[:H], W_upd[H:H + 1], b_upd.reshape(1, H), gamma.reshape(1, H),
      beta_eff)
    return out
